# baseline (speedup 1.0000x reference)
import functools

import jax
import jax.numpy as jnp
from jax import lax
from jax.experimental import pallas as pl
from jax.experimental.pallas import tpu as pltpu
from jax.experimental.pallas import tpu_sc as plsc

F32 = jnp.float32
I32 = jnp.int32

N = 10000
E = 320000
F = 128
EPT = E // 16
CH = 80
NCHUNK = EPT // CH
RPT = N // 16
ZR = 25
BN = 2000
BE = 4000


def _full(shape):
    return pl.BlockSpec(shape, lambda *_: tuple(0 for _ in shape))


def _proj_nodes_body(x_ref, ast_ref, adt_ref, a2p_ref, u_ref, v_ref, pn_ref,
                     qn_ref):
    x = x_ref[...]
    u = jnp.dot(x, ast_ref[...], preferred_element_type=F32)
    v = jnp.dot(x, adt_ref[...], preferred_element_type=F32)
    u_ref[...] = u
    v_ref[...] = v
    pn_ref[...] = jnp.dot(u, a2p_ref[...], preferred_element_type=F32)
    qn_ref[...] = jnp.dot(v, a2p_ref[...], preferred_element_type=F32)


def _proj_nodes(x, ast, adt, a2p):
    return pl.pallas_call(
        _proj_nodes_body,
        grid=(N // BN,),
        in_specs=[
            pl.BlockSpec((BN, F), lambda i: (i, 0)),
            _full((F, 128)), _full((F, 128)), _full((F, 16)),
        ],
        out_specs=[
            pl.BlockSpec((BN, 128), lambda i: (i, 0)),
            pl.BlockSpec((BN, 128), lambda i: (i, 0)),
            pl.BlockSpec((BN, 16), lambda i: (i, 0)),
            pl.BlockSpec((BN, 16), lambda i: (i, 0)),
        ],
        out_shape=[
            jax.ShapeDtypeStruct((N, 128), F32),
            jax.ShapeDtypeStruct((N, 128), F32),
            jax.ShapeDtypeStruct((N, 16), F32),
            jax.ShapeDtypeStruct((N, 16), F32),
        ],
    )(x, ast, adt, a2p)


def _ree_body(ee_ref, ar_ref, out_ref):
    ee = ee_ref[...]
    r = jnp.dot(ee, ar_ref[...], preferred_element_type=F32)
    out_ref[...] = jnp.concatenate([r, ee], axis=1)


def _ree(ee, ar16):
    return pl.pallas_call(
        _ree_body,
        grid=(E // BE,),
        in_specs=[pl.BlockSpec((BE, 16), lambda i: (i, 0)), _full((16, 16))],
        out_specs=pl.BlockSpec((BE, 32), lambda i: (i, 0)),
        out_shape=jax.ShapeDtypeStruct((E, 32), F32),
    )(ee, ar16)


def _relproj_body(rel_ref, w1_ref, a2rt_ref, a2p2_ref, rp_ref, rr_ref):
    t = jnp.dot(rel_ref[...], w1_ref[...], preferred_element_type=F32)
    rp = jnp.dot(t, a2rt_ref[...], preferred_element_type=F32)
    rp_ref[...] = rp
    rr_ref[...] = jnp.dot(rp, a2p2_ref[...], preferred_element_type=F32)


def _relproj(rel, w1, a2rt, a2p2):
    nr = rel.shape[0]
    return pl.pallas_call(
        _relproj_body,
        in_specs=[_full((nr, 16)), _full((16, 128)), _full((128, 128)),
                  _full((128, 16))],
        out_specs=[_full((nr, 128)), _full((nr, 16))],
        out_shape=[jax.ShapeDtypeStruct((nr, 128), F32),
                   jax.ShapeDtypeStruct((nr, 16), F32)],
    )(rel, w1, a2rt, a2p2)


def _combine1_body(u_ref, m0_ref, m1_ref, g0_ref, g1_ref, rs0_ref, rs1_ref,
                   k0_ref, k1_ref, a2st_ref, a2dt_ref, a2p2_ref,
                   u2_ref, v2_ref, p2_ref, q2_ref):
    m = jnp.concatenate(
        [m0_ref[...] + jnp.dot(g0_ref[...], k0_ref[...],
                               preferred_element_type=F32),
         m1_ref[...] + jnp.dot(g1_ref[...], k1_ref[...],
                               preferred_element_type=F32)], axis=1)
    rse = jnp.concatenate(
        [jnp.broadcast_to(rs0_ref[...][:, 0:1], (BN, 64)),
         jnp.broadcast_to(rs1_ref[...][:, 1:2], (BN, 64))], axis=1)
    rsc = jnp.where(rse == 0.0, 1e-12, rse)
    x2 = (u_ref[...] * rse + m) / rsc
    x2 = jnp.where(x2 > 0.0, x2, jnp.exp(jnp.minimum(x2, 0.0)) - 1.0)
    u2 = jnp.dot(x2, a2st_ref[...], preferred_element_type=F32)
    v2 = jnp.dot(x2, a2dt_ref[...], preferred_element_type=F32)
    u2_ref[...] = u2
    v2_ref[...] = v2
    p2_ref[...] = jnp.dot(u2, a2p2_ref[...], preferred_element_type=F32)
    q2_ref[...] = jnp.dot(v2, a2p2_ref[...], preferred_element_type=F32)


def _combine1(u, m0, m1, g0, g1, rs0, rs1, k0, k1, a2st, a2dt, a2p2):
    nb = pl.BlockSpec((BN, 128), lambda i: (i, 0))
    hb = pl.BlockSpec((BN, 64), lambda i: (i, 0))
    sb = pl.BlockSpec((BN, 16), lambda i: (i, 0))
    return pl.pallas_call(
        _combine1_body,
        grid=(N // BN,),
        in_specs=[nb, hb, hb, sb, sb, sb, sb, _full((16, 64)),
                  _full((16, 64)), _full((128, 128)), _full((128, 128)),
                  _full((128, 16))],
        out_specs=[nb, nb, sb, sb],
        out_shape=[jax.ShapeDtypeStruct((N, 128), F32),
                   jax.ShapeDtypeStruct((N, 128), F32),
                   jax.ShapeDtypeStruct((N, 16), F32),
                   jax.ShapeDtypeStruct((N, 16), F32)],
    )(u, m0, m1, g0, g1, rs0, rs1, k0, k1, a2st, a2dt, a2p2)


def _final_body(u2_ref, m2a_ref, m2b_ref, rsa_ref, out_ref):
    m = jnp.concatenate([m2a_ref[...], m2b_ref[...]], axis=1)
    rse = jnp.broadcast_to(rsa_ref[...][:, 0:1], (BN, 128))
    rsc = jnp.where(rse == 0.0, 1e-12, rse)
    x = (u2_ref[...] * rse + m) / rsc
    out_ref[...] = jnp.where(x > 0.0, x, jnp.exp(jnp.minimum(x, 0.0)) - 1.0)


def _final(u2, m2a, m2b, rsa):
    nb = pl.BlockSpec((BN, 128), lambda i: (i, 0))
    hb = pl.BlockSpec((BN, 64), lambda i: (i, 0))
    sb = pl.BlockSpec((BN, 16), lambda i: (i, 0))
    return pl.pallas_call(
        _final_body,
        grid=(N // BN,),
        in_specs=[nb, hb, hb, sb],
        out_specs=nb,
        out_shape=jax.ShapeDtypeStruct((N, 128), F32),
    )(u2, m2a, m2b, rsa)


_MESH = dict(core_axis_name="c", subcore_axis_name="s")


def _zero_fill(zb, width):
    @pl.loop(0, ZR)
    def zrow(rr):
        z = jnp.zeros((16,), F32)
        for k in range(width // 16):
            zb[rr, pl.ds(k * 16, 16)] = z


def _zero_acc(zbuf, sw, s, acc_sp, width, col0=0):
    @pl.loop(0, RPT // ZR)
    def zcp(k):
        base = s * RPT + k * ZR
        for w in range(width // sw):
            pltpu.sync_copy(zbuf, acc_sp.at[pl.ds(base, ZR),
                                            pl.ds(col0 + w * sw, sw)])


def _splat(val):
    return jnp.full((16,), val, I32)


def _adjust_idx(dstv, dv2, coff):
    @pl.loop(0, CH // 16)
    def adj(k):
        sl = pl.ds(k * 16, 16)
        dv2[sl] = dstv[sl] + coff


def _sc_layer1(src, dst, ree, pn, qn, v1s):
    mesh = plsc.VectorSubcoreMesh(**_MESH)
    nbuf = [
        pltpu.VMEM((CH,), I32),
        pltpu.VMEM((CH,), I32),
        pltpu.VMEM((CH,), I32),
        pltpu.VMEM((CH, 16), F32),
        pltpu.VMEM((CH, 16), F32),
        pltpu.VMEM((CH, 32), F32),
        pltpu.VMEM((CH, 64), F32),
    ]

    @functools.partial(
        pl.kernel,
        out_type=[jax.ShapeDtypeStruct((2, N, 96), F32)],
        mesh=mesh,
        compiler_params=pltpu.CompilerParams(needs_layout_passes=False,
                                             use_tc_tiling_on_sc=False),
        scratch_types=[
            pltpu.VMEM_SHARED((N, 96), F32),
        ] + nbuf + nbuf + [
            pltpu.VMEM((CH, 96), F32),
            pltpu.VMEM((CH * 16,), F32),
            pltpu.VMEM((ZR, 16), F32),
            pltpu.SemaphoreType.DMA,
            pltpu.SemaphoreType.DMA,
            pltpu.SemaphoreType.DMA,
            pltpu.SemaphoreType.DMA,
        ],
    )
    def body(src_h, dst_h, ree_h, pn_h, qn_h, v1s_h,
             acc_out,
             acc_sp,
             srcA, dstA, dv2A, gpA, gqA, reA, vrA,
             srcB, dstB, dv2B, gpB, gqB, reB, vrB,
             crows, ef, zb16,
             slA, slB, sgA, sgB):
        c = lax.axis_index("c")
        s = lax.axis_index("s")
        coff = c * N
        _zero_fill(zb16, 16)
        _zero_acc(zb16, 16, s, acc_sp, 96)
        plsc.subcore_barrier()

        def lin_issue(i, bufs, sem):
            off = s * EPT + jnp.minimum(i, NCHUNK - 1) * CH
            pltpu.async_copy(src_h.at[pl.ds(off, CH)], bufs[0], sem)
            pltpu.async_copy(dst_h.at[pl.ds(off, CH)], bufs[1], sem)
            pltpu.async_copy(ree_h.at[pl.ds(off, CH)], bufs[5], sem)

        def lin_wait(bufs, sem):
            pltpu.make_async_copy(src_h.at[pl.ds(0, CH)], bufs[0], sem).wait()
            pltpu.make_async_copy(dst_h.at[pl.ds(0, CH)], bufs[1], sem).wait()
            pltpu.make_async_copy(ree_h.at[pl.ds(0, CH)], bufs[5], sem).wait()

        def gath_issue(bufs, sem):
            _adjust_idx(bufs[1], bufs[2], coff)
            pltpu.async_copy(pn_h.at[bufs[0]], bufs[3], sem)
            pltpu.async_copy(qn_h.at[bufs[1]], bufs[4], sem)
            pltpu.async_copy(v1s_h.at[bufs[2]], bufs[6], sem)

        def gath_wait(bufs, sem):
            pltpu.make_async_copy(pn_h.at[bufs[0]], bufs[3], sem).wait()
            pltpu.make_async_copy(qn_h.at[bufs[1]], bufs[4], sem).wait()
            pltpu.make_async_copy(v1s_h.at[bufs[2]], bufs[6], sem).wait()

        def compute(bufs):
            srcv, dstv, dv2, gpv, gqv, reev, vrowsv = bufs

            @pl.loop(0, CH, unroll=4)
            def ecomp(j):
                sv = gpv[j] + gqv[j] + reev[j, pl.ds(0, 16)]
                e = jnp.exp(-jnp.maximum(sv, 0.2 * sv))
                crows[j, pl.ds(64, 16)] = e
                ef[pl.ds(j * 16, 16)] = e

            @pl.loop(0, CH, unroll=4)
            def tcomp(cc):
                eh = plsc.load_gather(ef, [_splat(cc * 16) + c])
                eerow = reev[cc, pl.ds(16, 16)]
                crows[cc, pl.ds(80, 16)] = eh * eerow
                for jj in range(4):
                    sl = pl.ds(jj * 16, 16)
                    crows[cc, sl] = eh * vrowsv[cc, sl]

            pltpu.sync_copy(crows, acc_sp.at[srcv], add=True)

        A = (srcA, dstA, dv2A, gpA, gqA, reA, vrA)
        B = (srcB, dstB, dv2B, gpB, gqB, reB, vrB)

        lin_issue(0, A, slA)
        lin_wait(A, slA)
        gath_issue(A, sgA)
        lin_issue(1, B, slB)

        @pl.loop(0, NCHUNK // 2)
        def it(k):
            lin_wait(B, slB)
            gath_issue(B, sgB)
            gath_wait(A, sgA)
            compute(A)
            lin_issue(2 * k + 2, A, slA)
            lin_wait(A, slA)
            gath_issue(A, sgA)
            gath_wait(B, sgB)
            compute(B)
            lin_issue(2 * k + 3, B, slB)

        gath_wait(A, sgA)
        if NCHUNK % 2 == 1:
            compute(A)
        lin_wait(B, slB)

        plsc.subcore_barrier()

        @pl.when(s == 0)
        def _():
            pltpu.sync_copy(acc_sp, acc_out.at[c])

    return body(src, dst, ree, pn, qn, v1s)


def _sc_layer2(src, dst, etyp, p2, q2, v2s, relall):
    mesh = plsc.VectorSubcoreMesh(**_MESH)
    nbuf = [
        pltpu.VMEM((CH,), I32),
        pltpu.VMEM((CH,), I32),
        pltpu.VMEM((CH,), I32),
        pltpu.VMEM((CH,), I32),
        pltpu.VMEM((CH, 16), F32),
        pltpu.VMEM((CH, 16), F32),
        pltpu.VMEM((CH, 64), F32),
    ]

    @functools.partial(
        pl.kernel,
        out_type=[jax.ShapeDtypeStruct((2, N, 80), F32)],
        mesh=mesh,
        compiler_params=pltpu.CompilerParams(needs_layout_passes=False,
                                             use_tc_tiling_on_sc=False),
        scratch_types=[
            pltpu.VMEM_SHARED((N, 80), F32),
            pltpu.VMEM((200 * 64 + 200,), F32),
        ] + nbuf + nbuf + [
            pltpu.VMEM((CH, 80), F32),
            pltpu.VMEM((CH * 16,), F32),
            pltpu.VMEM((ZR, 16), F32),
            pltpu.SemaphoreType.DMA,
            pltpu.SemaphoreType.DMA,
            pltpu.SemaphoreType.DMA,
            pltpu.SemaphoreType.DMA,
        ],
    )
    def body(src_h, dst_h, typ_h, p2_h, q2_h, v2s_h, rel_h,
             acc_out,
             acc_sp, relv,
             srcA, dstA, dv2A, typA, gpA, gqA, vrA,
             srcB, dstB, dv2B, typB, gpB, gqB, vrB,
             crows, ef, zb16,
             slA, slB, sgA, sgB):
        c = lax.axis_index("c")
        s = lax.axis_index("s")
        coff = c * N
        _zero_fill(zb16, 16)

        @pl.loop(0, 16)
        def rcp(k):
            pltpu.sync_copy(rel_h.at[pl.ds(c * 12800 + k * 800, 800)],
                            relv.at[pl.ds(k * 800, 800)])

        pltpu.sync_copy(rel_h.at[pl.ds(25600, 200)],
                        relv.at[pl.ds(12800, 200)])
        _zero_acc(zb16, 16, s, acc_sp, 80)
        plsc.subcore_barrier()

        def lin_issue(i, bufs, sem):
            off = s * EPT + jnp.minimum(i, NCHUNK - 1) * CH
            pltpu.async_copy(src_h.at[pl.ds(off, CH)], bufs[0], sem)
            pltpu.async_copy(dst_h.at[pl.ds(off, CH)], bufs[1], sem)
            pltpu.async_copy(typ_h.at[pl.ds(off, CH)], bufs[3], sem)

        def lin_wait(bufs, sem):
            pltpu.make_async_copy(src_h.at[pl.ds(0, CH)], bufs[0], sem).wait()
            pltpu.make_async_copy(dst_h.at[pl.ds(0, CH)], bufs[1], sem).wait()
            pltpu.make_async_copy(typ_h.at[pl.ds(0, CH)], bufs[3], sem).wait()

        def gath_issue(bufs, sem):
            _adjust_idx(bufs[1], bufs[2], coff)
            pltpu.async_copy(p2_h.at[bufs[0]], bufs[4], sem)
            pltpu.async_copy(q2_h.at[bufs[1]], bufs[5], sem)
            pltpu.async_copy(v2s_h.at[bufs[2]], bufs[6], sem)

        def gath_wait(bufs, sem):
            pltpu.make_async_copy(p2_h.at[bufs[0]], bufs[4], sem).wait()
            pltpu.make_async_copy(q2_h.at[bufs[1]], bufs[5], sem).wait()
            pltpu.make_async_copy(v2s_h.at[bufs[2]], bufs[6], sem).wait()

        lanes = lax.iota(I32, 16)

        def compute(bufs):
            srcv, dstv, dv2, typv, gpv, gqv, vrowsv = bufs

            @pl.loop(0, CH, unroll=4)
            def ecomp(j):
                tj = plsc.load_gather(typv, [_splat(j)])
                rr = plsc.load_gather(relv, [tj + 200 * 64])
                sv = gpv[j] + gqv[j] + rr
                e = jnp.exp(-jnp.maximum(sv, 0.2 * sv))
                crows[j, pl.ds(64, 16)] = e
                ef[pl.ds(j * 16, 16)] = e

            @pl.loop(0, CH, unroll=4)
            def tcomp(cc):
                e0 = plsc.load_gather(ef, [_splat(cc * 16)])
                tvec = plsc.load_gather(typv, [_splat(cc)])
                tbase = tvec * 64
                for jj in range(4):
                    sl = pl.ds(jj * 16, 16)
                    w2j = plsc.load_gather(relv, [tbase + lanes + jj * 16])
                    crows[cc, sl] = e0 * (vrowsv[cc, sl] + w2j)

            pltpu.sync_copy(crows, acc_sp.at[srcv], add=True)

        A = (srcA, dstA, dv2A, typA, gpA, gqA, vrA)
        B = (srcB, dstB, dv2B, typB, gpB, gqB, vrB)

        lin_issue(0, A, slA)
        lin_wait(A, slA)
        gath_issue(A, sgA)
        lin_issue(1, B, slB)

        @pl.loop(0, NCHUNK // 2)
        def it(k):
            lin_wait(B, slB)
            gath_issue(B, sgB)
            gath_wait(A, sgA)
            compute(A)
            lin_issue(2 * k + 2, A, slA)
            lin_wait(A, slA)
            gath_issue(A, sgA)
            gath_wait(B, sgB)
            compute(B)
            lin_issue(2 * k + 3, B, slB)

        gath_wait(A, sgA)
        if NCHUNK % 2 == 1:
            compute(A)
        lin_wait(B, slB)

        plsc.subcore_barrier()

        @pl.when(s == 0)
        def _():
            pltpu.sync_copy(acc_sp, acc_out.at[c])

    return body(src, dst, etyp, p2, q2, v2s, relall)


def kernel(Corpus_, batch_inputs, unique_entity_embed, unique_relation_embed,
           edge_list, edge_type, edge_embed, a0, a2_0, a1, a2_1, W_1, a_out,
           a2_out):
    x = unique_entity_embed.astype(F32)
    ee = edge_embed.astype(F32)
    src = edge_list[0].astype(I32)
    dst = edge_list[1].astype(I32)
    etyp = edge_type.astype(I32)

    ast = jnp.concatenate([a0[:, :F], a1[:, :F]], axis=0).T
    adt = jnp.concatenate([a0[:, F:2 * F], a1[:, F:2 * F]], axis=0).T
    a2p = jnp.zeros((128, 16), F32)
    a2p = a2p.at[0:64, 0].set(a2_0[0]).at[64:128, 1].set(a2_1[0])
    ar0 = (a2_0 @ a0[:, 2 * F:2 * F + 16])[0]
    ar1 = (a2_1 @ a1[:, 2 * F:2 * F + 16])[0]
    ar16 = jnp.zeros((16, 16), F32).at[:, 0].set(ar0).at[:, 1].set(ar1)
    k0 = a0[:, 2 * F:2 * F + 16].T
    k1 = a1[:, 2 * F:2 * F + 16].T
    a2st = a_out[:, :128].T
    a2dt = a_out[:, 128:256].T
    a2rt = a_out[:, 256:384].T
    a2p2 = jnp.zeros((128, 16), F32).at[:, 0].set(a2_out[0])

    u, v1, pn, qn = _proj_nodes(x, ast, adt, a2p)
    v1s = jnp.concatenate([v1[:, 0:64], v1[:, 64:128]], axis=0)
    ree = _ree(ee, ar16)

    acc1, = _sc_layer1(src, dst, ree, pn, qn, v1s)
    m0 = acc1[0, :, 0:64]
    m1 = acc1[1, :, 0:64]
    rs0 = acc1[0, :, 64:80]
    rs1 = acc1[1, :, 64:80]
    g0 = acc1[0, :, 80:96]
    g1 = acc1[1, :, 80:96]

    relproj, relr = _relproj(unique_relation_embed.astype(F32), W_1, a2rt,
                             a2p2)
    relall = jnp.concatenate([relproj[:, 0:64].reshape(-1),
                              relproj[:, 64:128].reshape(-1), relr[:, 0]])

    u2, v2, p2, q2 = _combine1(u, m0, m1, g0, g1, rs0, rs1, k0, k1,
                               a2st, a2dt, a2p2)
    v2s = jnp.concatenate([v2[:, 0:64], v2[:, 64:128]], axis=0)

    acc2, = _sc_layer2(src, dst, etyp, p2, q2, v2s, relall)

    return _final(u2, acc2[0, :, 0:64], acc2[1, :, 0:64], acc2[0, :, 64:80])


# speedup vs baseline: 1.4185x; 1.4185x over previous
import functools

import jax
import jax.numpy as jnp
from jax import lax
from jax.experimental import pallas as pl
from jax.experimental.pallas import tpu as pltpu
from jax.experimental.pallas import tpu_sc as plsc

F32 = jnp.float32
I32 = jnp.int32

N = 10000
E = 320000
F = 128
NW = 32
EPW = E // NW
CH = 40
NCHUNK = EPW // CH
RPT = N // 16
ZR = 25
BN = 2000
BE = 4000


def _full(shape):
    return pl.BlockSpec(shape, lambda *_: tuple(0 for _ in shape))


def _proj_nodes_body(x_ref, ast_ref, adt_ref, a2p_ref, u_ref, v_ref, pn_ref,
                     qn_ref):
    x = x_ref[...]
    u = jnp.dot(x, ast_ref[...], preferred_element_type=F32)
    v = jnp.dot(x, adt_ref[...], preferred_element_type=F32)
    u_ref[...] = u
    v_ref[...] = v
    pn_ref[...] = jnp.dot(u, a2p_ref[...], preferred_element_type=F32)
    qn_ref[...] = jnp.dot(v, a2p_ref[...], preferred_element_type=F32)


def _proj_nodes(x, ast, adt, a2p):
    return pl.pallas_call(
        _proj_nodes_body,
        grid=(N // BN,),
        in_specs=[
            pl.BlockSpec((BN, F), lambda i: (i, 0)),
            _full((F, 128)), _full((F, 128)), _full((F, 16)),
        ],
        out_specs=[
            pl.BlockSpec((BN, 128), lambda i: (i, 0)),
            pl.BlockSpec((BN, 128), lambda i: (i, 0)),
            pl.BlockSpec((BN, 16), lambda i: (i, 0)),
            pl.BlockSpec((BN, 16), lambda i: (i, 0)),
        ],
        out_shape=[
            jax.ShapeDtypeStruct((N, 128), F32),
            jax.ShapeDtypeStruct((N, 128), F32),
            jax.ShapeDtypeStruct((N, 16), F32),
            jax.ShapeDtypeStruct((N, 16), F32),
        ],
    )(x, ast, adt, a2p)


def _ree_body(ee_ref, ar_ref, out_ref):
    ee = ee_ref[...]
    r = jnp.dot(ee, ar_ref[...], preferred_element_type=F32)
    out_ref[...] = jnp.concatenate([r, ee], axis=1)


def _ree(ee, ar16):
    return pl.pallas_call(
        _ree_body,
        grid=(E // BE,),
        in_specs=[pl.BlockSpec((BE, 16), lambda i: (i, 0)), _full((16, 16))],
        out_specs=pl.BlockSpec((BE, 32), lambda i: (i, 0)),
        out_shape=jax.ShapeDtypeStruct((E, 32), F32),
    )(ee, ar16)


def _relproj_body(rel_ref, w1_ref, a2rt_ref, a2p2_ref, rp_ref, rr_ref):
    t = jnp.dot(rel_ref[...], w1_ref[...], preferred_element_type=F32)
    rp = jnp.dot(t, a2rt_ref[...], preferred_element_type=F32)
    rp_ref[...] = rp
    rr_ref[...] = jnp.dot(rp, a2p2_ref[...], preferred_element_type=F32)


def _relproj(rel, w1, a2rt, a2p2):
    nr = rel.shape[0]
    return pl.pallas_call(
        _relproj_body,
        in_specs=[_full((nr, 16)), _full((16, 128)), _full((128, 128)),
                  _full((128, 16))],
        out_specs=[_full((nr, 128)), _full((nr, 16))],
        out_shape=[jax.ShapeDtypeStruct((nr, 128), F32),
                   jax.ShapeDtypeStruct((nr, 16), F32)],
    )(rel, w1, a2rt, a2p2)


def _combine1_body(u_ref, m1a_ref, m1b_ref, ga_ref, gb_ref, rsa_ref, rsb_ref,
                   k_ref, a2st_ref, a2dt_ref, a2p2_ref,
                   u2_ref, v2_ref, p2_ref, q2_ref):
    m = m1a_ref[0] + m1b_ref[0] + jnp.dot(
        ga_ref[0] + gb_ref[0], k_ref[...], preferred_element_type=F32)
    rs = rsa_ref[0] + rsb_ref[0]
    rse = jnp.concatenate(
        [jnp.broadcast_to(rs[:, 0:1], (BN, 64)),
         jnp.broadcast_to(rs[:, 1:2], (BN, 64))], axis=1)
    rsc = jnp.where(rse == 0.0, 1e-12, rse)
    x2 = (u_ref[...] * rse + m) / rsc
    x2 = jnp.where(x2 > 0.0, x2, jnp.exp(jnp.minimum(x2, 0.0)) - 1.0)
    u2 = jnp.dot(x2, a2st_ref[...], preferred_element_type=F32)
    v2 = jnp.dot(x2, a2dt_ref[...], preferred_element_type=F32)
    u2_ref[...] = u2
    v2_ref[...] = v2
    p2_ref[...] = jnp.dot(u2, a2p2_ref[...], preferred_element_type=F32)
    q2_ref[...] = jnp.dot(v2, a2p2_ref[...], preferred_element_type=F32)


def _combine1(u, m1, g, rs1, k, a2st, a2dt, a2p2):
    nb = pl.BlockSpec((BN, 128), lambda i: (i, 0))
    sb = pl.BlockSpec((BN, 16), lambda i: (i, 0))

    def _core(which):
        return lambda i: (which, i, 0)

    return pl.pallas_call(
        _combine1_body,
        grid=(N // BN,),
        in_specs=[nb,
                  pl.BlockSpec((1, BN, 128), _core(0)),
                  pl.BlockSpec((1, BN, 128), _core(1)),
                  pl.BlockSpec((1, BN, 32), _core(0)),
                  pl.BlockSpec((1, BN, 32), _core(1)),
                  pl.BlockSpec((1, BN, 16), _core(0)),
                  pl.BlockSpec((1, BN, 16), _core(1)),
                  _full((32, 128)),
                  _full((128, 128)), _full((128, 128)), _full((128, 16))],
        out_specs=[nb, nb, sb, sb],
        out_shape=[jax.ShapeDtypeStruct((N, 128), F32),
                   jax.ShapeDtypeStruct((N, 128), F32),
                   jax.ShapeDtypeStruct((N, 16), F32),
                   jax.ShapeDtypeStruct((N, 16), F32)],
    )(u, m1, m1, g, g, rs1, rs1, k, a2st, a2dt, a2p2)


def _final_body(u2_ref, m2a_ref, m2b_ref, rsa_ref, rsb_ref, out_ref):
    m = m2a_ref[0] + m2b_ref[0]
    rs = rsa_ref[0] + rsb_ref[0]
    rse = jnp.broadcast_to(rs[:, 0:1], (BN, 128))
    rsc = jnp.where(rse == 0.0, 1e-12, rse)
    x = (u2_ref[...] * rse + m) / rsc
    out_ref[...] = jnp.where(x > 0.0, x, jnp.exp(jnp.minimum(x, 0.0)) - 1.0)


def _final(u2, m2, rs2):
    nb = pl.BlockSpec((BN, 128), lambda i: (i, 0))

    def _core(which):
        return lambda i: (which, i, 0)

    return pl.pallas_call(
        _final_body,
        grid=(N // BN,),
        in_specs=[nb,
                  pl.BlockSpec((1, BN, 128), _core(0)),
                  pl.BlockSpec((1, BN, 128), _core(1)),
                  pl.BlockSpec((1, BN, 16), _core(0)),
                  pl.BlockSpec((1, BN, 16), _core(1))],
        out_specs=nb,
        out_shape=jax.ShapeDtypeStruct((N, 128), F32),
    )(u2, m2, m2, rs2, rs2)


_MESH = dict(core_axis_name="c", subcore_axis_name="s")


def _zero_fill(zb, width):
    @pl.loop(0, ZR)
    def zrow(rr):
        z = jnp.zeros((16,), F32)
        for k in range(width // 16):
            zb[rr, pl.ds(k * 16, 16)] = z


def _zero_acc(zbuf, sw, s, acc_sp, width):
    @pl.loop(0, RPT // ZR)
    def zcp(k):
        base = s * RPT + k * ZR
        for w in range(width // sw):
            pltpu.sync_copy(zbuf, acc_sp.at[pl.ds(base, ZR),
                                            pl.ds(w * sw, sw)])


def _splat(val):
    return jnp.full((16,), val, I32)


def _sc_layer1(src, dst, ree, pn, qn, v1):
    mesh = plsc.VectorSubcoreMesh(**_MESH)
    nbuf = [
        pltpu.VMEM((CH,), I32),
        pltpu.VMEM((CH,), I32),
        pltpu.VMEM((CH, 16), F32),
        pltpu.VMEM((CH, 16), F32),
        pltpu.VMEM((CH, 32), F32),
        pltpu.VMEM((CH, 128), F32),
    ]

    @functools.partial(
        pl.kernel,
        out_type=[jax.ShapeDtypeStruct((2, N, 128), F32),
                  jax.ShapeDtypeStruct((2, N, 16), F32),
                  jax.ShapeDtypeStruct((2, N, 32), F32)],
        mesh=mesh,
        compiler_params=pltpu.CompilerParams(needs_layout_passes=False,
                                             use_tc_tiling_on_sc=False),
        scratch_types=[
            pltpu.VMEM_SHARED((N, 128), F32),
            pltpu.VMEM_SHARED((N, 16), F32),
            pltpu.VMEM_SHARED((N, 32), F32),
        ] + nbuf + nbuf + [
            pltpu.VMEM((CH, 16), F32),
            pltpu.VMEM((CH * 16,), F32),
            pltpu.VMEM((CH, 32), F32),
            pltpu.VMEM((ZR, 64), F32),
            pltpu.VMEM((ZR, 16), F32),
            pltpu.SemaphoreType.DMA,
            pltpu.SemaphoreType.DMA,
            pltpu.SemaphoreType.DMA,
            pltpu.SemaphoreType.DMA,
        ],
    )
    def body(src_h, dst_h, ree_h, pn_h, qn_h, v1_h,
             m_out, rs_out, g_out,
             m_sp, rs_sp, g_sp,
             srcA, dstA, gpA, gqA, reA, vrA,
             srcB, dstB, gpB, gqB, reB, vrB,
             ev, ef, growsv, zb64, zb16,
             slA, slB, sgA, sgB):
        c = lax.axis_index("c")
        s = lax.axis_index("s")
        wid = s * 2 + c
        _zero_fill(zb64, 64)
        _zero_fill(zb16, 16)
        _zero_acc(zb64, 64, s, m_sp, 128)
        _zero_acc(zb16, 16, s, rs_sp, 16)
        _zero_acc(zb16, 16, s, g_sp, 32)
        plsc.subcore_barrier()

        def lin_issue(i, bufs, sem):
            off = wid * EPW + jnp.minimum(i, NCHUNK - 1) * CH
            pltpu.async_copy(src_h.at[pl.ds(off, CH)], bufs[0], sem)
            pltpu.async_copy(dst_h.at[pl.ds(off, CH)], bufs[1], sem)
            pltpu.async_copy(ree_h.at[pl.ds(off, CH)], bufs[4], sem)

        def lin_wait(bufs, sem):
            pltpu.make_async_copy(src_h.at[pl.ds(0, CH)], bufs[0], sem).wait()
            pltpu.make_async_copy(dst_h.at[pl.ds(0, CH)], bufs[1], sem).wait()
            pltpu.make_async_copy(ree_h.at[pl.ds(0, CH)], bufs[4], sem).wait()

        def gath_issue(bufs, sem):
            pltpu.async_copy(pn_h.at[bufs[0]], bufs[2], sem)
            pltpu.async_copy(qn_h.at[bufs[1]], bufs[3], sem)
            pltpu.async_copy(v1_h.at[bufs[1]], bufs[5], sem)

        def gath_wait(bufs, sem):
            pltpu.make_async_copy(pn_h.at[bufs[0]], bufs[2], sem).wait()
            pltpu.make_async_copy(qn_h.at[bufs[1]], bufs[3], sem).wait()
            pltpu.make_async_copy(v1_h.at[bufs[1]], bufs[5], sem).wait()

        def compute(bufs):
            srcv, dstv, gpv, gqv, reev, vrowsv = bufs

            @pl.loop(0, CH, unroll=4)
            def ecomp(j):
                sv = gpv[j] + gqv[j] + reev[j, pl.ds(0, 16)]
                e = jnp.exp(-jnp.maximum(sv, 0.2 * sv))
                ev[j] = e
                ef[pl.ds(j * 16, 16)] = e

            pltpu.sync_copy(ev, rs_sp.at[srcv], add=True)

            @pl.loop(0, CH, unroll=4)
            def tcomp(cc):
                e0 = plsc.load_gather(ef, [_splat(cc * 16)])
                e1 = plsc.load_gather(ef, [_splat(cc * 16 + 1)])
                eerow = reev[cc, pl.ds(16, 16)]
                growsv[cc, pl.ds(0, 16)] = e0 * eerow
                growsv[cc, pl.ds(16, 16)] = e1 * eerow
                for jj in range(8):
                    eh = e0 if jj < 4 else e1
                    sl = pl.ds(jj * 16, 16)
                    vrowsv[cc, sl] = eh * vrowsv[cc, sl]

            pltpu.sync_copy(growsv, g_sp.at[srcv], add=True)
            pltpu.sync_copy(vrowsv, m_sp.at[srcv], add=True)

        A = (srcA, dstA, gpA, gqA, reA, vrA)
        B = (srcB, dstB, gpB, gqB, reB, vrB)

        lin_issue(0, A, slA)
        lin_wait(A, slA)
        gath_issue(A, sgA)
        lin_issue(1, B, slB)

        @pl.loop(0, NCHUNK // 2)
        def it(k):
            lin_wait(B, slB)
            gath_issue(B, sgB)
            gath_wait(A, sgA)
            compute(A)
            lin_issue(2 * k + 2, A, slA)
            lin_wait(A, slA)
            gath_issue(A, sgA)
            gath_wait(B, sgB)
            compute(B)
            lin_issue(2 * k + 3, B, slB)

        gath_wait(A, sgA)
        if NCHUNK % 2 == 1:
            compute(A)
        lin_wait(B, slB)

        plsc.subcore_barrier()

        @pl.when(s == 0)
        def _():
            pltpu.sync_copy(m_sp, m_out.at[c])
            pltpu.sync_copy(rs_sp, rs_out.at[c])
            pltpu.sync_copy(g_sp, g_out.at[c])

    return body(src, dst, ree, pn, qn, v1)


def _sc_layer2(src, dst, etyp, p2, q2, v2, relall):
    mesh = plsc.VectorSubcoreMesh(**_MESH)
    nbuf = [
        pltpu.VMEM((CH,), I32),
        pltpu.VMEM((CH,), I32),
        pltpu.VMEM((CH,), I32),
        pltpu.VMEM((CH, 16), F32),
        pltpu.VMEM((CH, 16), F32),
        pltpu.VMEM((CH, 128), F32),
    ]

    @functools.partial(
        pl.kernel,
        out_type=[jax.ShapeDtypeStruct((2, N, 128), F32),
                  jax.ShapeDtypeStruct((2, N, 16), F32)],
        mesh=mesh,
        compiler_params=pltpu.CompilerParams(needs_layout_passes=False,
                                             use_tc_tiling_on_sc=False),
        scratch_types=[
            pltpu.VMEM_SHARED((N, 128), F32),
            pltpu.VMEM_SHARED((N, 16), F32),
            pltpu.VMEM((200 * 129,), F32),
        ] + nbuf + nbuf + [
            pltpu.VMEM((CH, 16), F32),
            pltpu.VMEM((CH * 16,), F32),
            pltpu.VMEM((ZR, 16), F32),
            pltpu.SemaphoreType.DMA,
            pltpu.SemaphoreType.DMA,
            pltpu.SemaphoreType.DMA,
            pltpu.SemaphoreType.DMA,
        ],
    )
    def body(src_h, dst_h, typ_h, p2_h, q2_h, v2_h, rp_h,
             m_out, rs_out,
             m_sp, rs_sp, relv,
             srcA, dstA, typA, gpA, gqA, vrA,
             srcB, dstB, typB, gpB, gqB, vrB,
             ev, ef, zb16,
             slA, slB, sgA, sgB):
        c = lax.axis_index("c")
        s = lax.axis_index("s")
        wid = s * 2 + c
        _zero_fill(zb16, 16)

        @pl.loop(0, 25)
        def rcp(k):
            sl = pl.ds(k * 1032, 1032)
            pltpu.sync_copy(rp_h.at[sl], relv.at[sl])

        _zero_acc(zb16, 16, s, m_sp, 128)
        _zero_acc(zb16, 16, s, rs_sp, 16)
        plsc.subcore_barrier()

        def lin_issue(i, bufs, sem):
            off = wid * EPW + jnp.minimum(i, NCHUNK - 1) * CH
            pltpu.async_copy(src_h.at[pl.ds(off, CH)], bufs[0], sem)
            pltpu.async_copy(dst_h.at[pl.ds(off, CH)], bufs[1], sem)
            pltpu.async_copy(typ_h.at[pl.ds(off, CH)], bufs[2], sem)

        def lin_wait(bufs, sem):
            pltpu.make_async_copy(src_h.at[pl.ds(0, CH)], bufs[0], sem).wait()
            pltpu.make_async_copy(dst_h.at[pl.ds(0, CH)], bufs[1], sem).wait()
            pltpu.make_async_copy(typ_h.at[pl.ds(0, CH)], bufs[2], sem).wait()

        def gath_issue(bufs, sem):
            pltpu.async_copy(p2_h.at[bufs[0]], bufs[3], sem)
            pltpu.async_copy(q2_h.at[bufs[1]], bufs[4], sem)
            pltpu.async_copy(v2_h.at[bufs[1]], bufs[5], sem)

        def gath_wait(bufs, sem):
            pltpu.make_async_copy(p2_h.at[bufs[0]], bufs[3], sem).wait()
            pltpu.make_async_copy(q2_h.at[bufs[1]], bufs[4], sem).wait()
            pltpu.make_async_copy(v2_h.at[bufs[1]], bufs[5], sem).wait()

        lanes = lax.iota(I32, 16)

        def compute(bufs):
            srcv, dstv, typv, gpv, gqv, vrowsv = bufs

            @pl.loop(0, CH, unroll=4)
            def ecomp(j):
                tj = plsc.load_gather(typv, [_splat(j)])
                rr = plsc.load_gather(relv, [tj + 200 * 128])
                sv = gpv[j] + gqv[j] + rr
                e = jnp.exp(-jnp.maximum(sv, 0.2 * sv))
                ev[j] = e
                ef[pl.ds(j * 16, 16)] = e

            pltpu.sync_copy(ev, rs_sp.at[srcv], add=True)

            @pl.loop(0, CH, unroll=4)
            def tcomp(cc):
                e0 = plsc.load_gather(ef, [_splat(cc * 16)])
                tvec = plsc.load_gather(typv, [_splat(cc)])
                tbase = tvec * 128
                for jj in range(8):
                    sl = pl.ds(jj * 16, 16)
                    w2j = plsc.load_gather(relv, [tbase + lanes + jj * 16])
                    vrowsv[cc, sl] = e0 * (vrowsv[cc, sl] + w2j)

            pltpu.sync_copy(vrowsv, m_sp.at[srcv], add=True)

        A = (srcA, dstA, typA, gpA, gqA, vrA)
        B = (srcB, dstB, typB, gpB, gqB, vrB)

        lin_issue(0, A, slA)
        lin_wait(A, slA)
        gath_issue(A, sgA)
        lin_issue(1, B, slB)

        @pl.loop(0, NCHUNK // 2)
        def it(k):
            lin_wait(B, slB)
            gath_issue(B, sgB)
            gath_wait(A, sgA)
            compute(A)
            lin_issue(2 * k + 2, A, slA)
            lin_wait(A, slA)
            gath_issue(A, sgA)
            gath_wait(B, sgB)
            compute(B)
            lin_issue(2 * k + 3, B, slB)

        gath_wait(A, sgA)
        if NCHUNK % 2 == 1:
            compute(A)
        lin_wait(B, slB)

        plsc.subcore_barrier()

        @pl.when(s == 0)
        def _():
            pltpu.sync_copy(m_sp, m_out.at[c])
            pltpu.sync_copy(rs_sp, rs_out.at[c])

    return body(src, dst, etyp, p2, q2, v2, relall)


def kernel(Corpus_, batch_inputs, unique_entity_embed, unique_relation_embed,
           edge_list, edge_type, edge_embed, a0, a2_0, a1, a2_1, W_1, a_out,
           a2_out):
    x = unique_entity_embed.astype(F32)
    ee = edge_embed.astype(F32)
    src = edge_list[0].astype(I32)
    dst = edge_list[1].astype(I32)
    etyp = edge_type.astype(I32)

    ast = jnp.concatenate([a0[:, :F], a1[:, :F]], axis=0).T
    adt = jnp.concatenate([a0[:, F:2 * F], a1[:, F:2 * F]], axis=0).T
    a2p = jnp.zeros((128, 16), F32)
    a2p = a2p.at[0:64, 0].set(a2_0[0]).at[64:128, 1].set(a2_1[0])
    ar0 = (a2_0 @ a0[:, 2 * F:2 * F + 16])[0]
    ar1 = (a2_1 @ a1[:, 2 * F:2 * F + 16])[0]
    ar16 = jnp.zeros((16, 16), F32).at[:, 0].set(ar0).at[:, 1].set(ar1)
    kmat = jnp.zeros((32, 128), F32)
    kmat = kmat.at[0:16, 0:64].set(a0[:, 2 * F:2 * F + 16].T)
    kmat = kmat.at[16:32, 64:128].set(a1[:, 2 * F:2 * F + 16].T)
    a2st = a_out[:, :128].T
    a2dt = a_out[:, 128:256].T
    a2rt = a_out[:, 256:384].T
    a2p2 = jnp.zeros((128, 16), F32).at[:, 0].set(a2_out[0])

    u, v1, pn, qn = _proj_nodes(x, ast, adt, a2p)
    ree = _ree(ee, ar16)

    m1, rs1, g = _sc_layer1(src, dst, ree, pn, qn, v1)

    relproj, relr = _relproj(unique_relation_embed.astype(F32), W_1, a2rt,
                             a2p2)
    relall = jnp.concatenate([relproj.reshape(-1), relr[:, 0]])

    u2, v2, p2, q2 = _combine1(u, m1, g, rs1, kmat, a2st, a2dt, a2p2)

    m2, rs2 = _sc_layer2(src, dst, etyp, p2, q2, v2, relall)

    return _final(u2, m2, rs2)


# speedup vs baseline: 1.4395x; 1.0149x over previous
import functools

import jax
import jax.numpy as jnp
from jax import lax
from jax.experimental import pallas as pl
from jax.experimental.pallas import tpu as pltpu
from jax.experimental.pallas import tpu_sc as plsc

F32 = jnp.float32
I32 = jnp.int32

N = 10000
E = 320000
F = 128
NW = 32
EPW = E // NW
CH = 40
NCHUNK = EPW // CH
RPT = N // 16
ZR = 25
BN = 2000
BE = 4000


def _full(shape):
    return pl.BlockSpec(shape, lambda *_: tuple(0 for _ in shape))


def _proj_nodes_body(x_ref, ast_ref, adt_ref, a2p_ref, u_ref, v_ref, pn_ref,
                     qn_ref):
    x = x_ref[...]
    u = jnp.dot(x, ast_ref[...], preferred_element_type=F32)
    v = jnp.dot(x, adt_ref[...], preferred_element_type=F32)
    u_ref[...] = u
    v_ref[...] = v
    pn_ref[...] = jnp.dot(u, a2p_ref[...], preferred_element_type=F32)
    qn_ref[...] = jnp.dot(v, a2p_ref[...], preferred_element_type=F32)


def _proj_nodes(x, ast, adt, a2p):
    return pl.pallas_call(
        _proj_nodes_body,
        grid=(N // BN,),
        in_specs=[
            pl.BlockSpec((BN, F), lambda i: (i, 0)),
            _full((F, 128)), _full((F, 128)), _full((F, 16)),
        ],
        out_specs=[
            pl.BlockSpec((BN, 128), lambda i: (i, 0)),
            pl.BlockSpec((BN, 128), lambda i: (i, 0)),
            pl.BlockSpec((BN, 16), lambda i: (i, 0)),
            pl.BlockSpec((BN, 16), lambda i: (i, 0)),
        ],
        out_shape=[
            jax.ShapeDtypeStruct((N, 128), F32),
            jax.ShapeDtypeStruct((N, 128), F32),
            jax.ShapeDtypeStruct((N, 16), F32),
            jax.ShapeDtypeStruct((N, 16), F32),
        ],
    )(x, ast, adt, a2p)


def _ree_body(ee_ref, ar_ref, out_ref):
    ee = ee_ref[...]
    r = jnp.dot(ee, ar_ref[...], preferred_element_type=F32)
    out_ref[...] = jnp.concatenate([r, ee], axis=1)


def _ree(ee, ar16):
    return pl.pallas_call(
        _ree_body,
        grid=(E // BE,),
        in_specs=[pl.BlockSpec((BE, 16), lambda i: (i, 0)), _full((16, 16))],
        out_specs=pl.BlockSpec((BE, 32), lambda i: (i, 0)),
        out_shape=jax.ShapeDtypeStruct((E, 32), F32),
    )(ee, ar16)


def _relproj_body(rel_ref, w1_ref, a2rt_ref, a2p2_ref, rp_ref, rr_ref):
    t = jnp.dot(rel_ref[...], w1_ref[...], preferred_element_type=F32)
    rp = jnp.dot(t, a2rt_ref[...], preferred_element_type=F32)
    rp_ref[...] = rp
    rr_ref[...] = jnp.dot(rp, a2p2_ref[...], preferred_element_type=F32)


def _relproj(rel, w1, a2rt, a2p2):
    nr = rel.shape[0]
    return pl.pallas_call(
        _relproj_body,
        in_specs=[_full((nr, 16)), _full((16, 128)), _full((128, 128)),
                  _full((128, 16))],
        out_specs=[_full((nr, 128)), _full((nr, 16))],
        out_shape=[jax.ShapeDtypeStruct((nr, 128), F32),
                   jax.ShapeDtypeStruct((nr, 16), F32)],
    )(rel, w1, a2rt, a2p2)


def _combine1_body(u_ref, m1a_ref, m1b_ref, rga_ref, rgb_ref,
                   k_ref, a2st_ref, a2dt_ref, a2p2_ref,
                   u2_ref, v2_ref, p2_ref, q2_ref):
    rga = rga_ref[0]
    rgb = rgb_ref[0]
    m = m1a_ref[0] + m1b_ref[0] + jnp.dot(
        rga[:, 16:48] + rgb[:, 16:48], k_ref[...],
        preferred_element_type=F32)
    rs = rga[:, 0:16] + rgb[:, 0:16]
    rse = jnp.concatenate(
        [jnp.broadcast_to(rs[:, 0:1], (BN, 64)),
         jnp.broadcast_to(rs[:, 1:2], (BN, 64))], axis=1)
    rsc = jnp.where(rse == 0.0, 1e-12, rse)
    x2 = (u_ref[...] * rse + m) / rsc
    x2 = jnp.where(x2 > 0.0, x2, jnp.exp(jnp.minimum(x2, 0.0)) - 1.0)
    u2 = jnp.dot(x2, a2st_ref[...], preferred_element_type=F32)
    v2 = jnp.dot(x2, a2dt_ref[...], preferred_element_type=F32)
    u2_ref[...] = u2
    v2_ref[...] = v2
    p2_ref[...] = jnp.dot(u2, a2p2_ref[...], preferred_element_type=F32)
    q2_ref[...] = jnp.dot(v2, a2p2_ref[...], preferred_element_type=F32)


def _combine1(u, m1, rg1, k, a2st, a2dt, a2p2):
    nb = pl.BlockSpec((BN, 128), lambda i: (i, 0))
    sb = pl.BlockSpec((BN, 16), lambda i: (i, 0))

    def _core(which):
        return lambda i: (which, i, 0)

    return pl.pallas_call(
        _combine1_body,
        grid=(N // BN,),
        in_specs=[nb,
                  pl.BlockSpec((1, BN, 128), _core(0)),
                  pl.BlockSpec((1, BN, 128), _core(1)),
                  pl.BlockSpec((1, BN, 48), _core(0)),
                  pl.BlockSpec((1, BN, 48), _core(1)),
                  _full((32, 128)),
                  _full((128, 128)), _full((128, 128)), _full((128, 16))],
        out_specs=[nb, nb, sb, sb],
        out_shape=[jax.ShapeDtypeStruct((N, 128), F32),
                   jax.ShapeDtypeStruct((N, 128), F32),
                   jax.ShapeDtypeStruct((N, 16), F32),
                   jax.ShapeDtypeStruct((N, 16), F32)],
    )(u, m1, m1, rg1, rg1, k, a2st, a2dt, a2p2)


def _final_body(u2_ref, m2a_ref, m2b_ref, rsa_ref, rsb_ref, out_ref):
    m = m2a_ref[0] + m2b_ref[0]
    rs = rsa_ref[0] + rsb_ref[0]
    rse = jnp.broadcast_to(rs[:, 0:1], (BN, 128))
    rsc = jnp.where(rse == 0.0, 1e-12, rse)
    x = (u2_ref[...] * rse + m) / rsc
    out_ref[...] = jnp.where(x > 0.0, x, jnp.exp(jnp.minimum(x, 0.0)) - 1.0)


def _final(u2, m2, rs2):
    nb = pl.BlockSpec((BN, 128), lambda i: (i, 0))

    def _core(which):
        return lambda i: (which, i, 0)

    return pl.pallas_call(
        _final_body,
        grid=(N // BN,),
        in_specs=[nb,
                  pl.BlockSpec((1, BN, 128), _core(0)),
                  pl.BlockSpec((1, BN, 128), _core(1)),
                  pl.BlockSpec((1, BN, 16), _core(0)),
                  pl.BlockSpec((1, BN, 16), _core(1))],
        out_specs=nb,
        out_shape=jax.ShapeDtypeStruct((N, 128), F32),
    )(u2, m2, m2, rs2, rs2)


_MESH = dict(core_axis_name="c", subcore_axis_name="s")


def _zero_fill(zb, width):
    @pl.loop(0, ZR)
    def zrow(rr):
        z = jnp.zeros((16,), F32)
        for k in range(width // 16):
            zb[rr, pl.ds(k * 16, 16)] = z


def _zero_acc(zbuf, sw, s, acc_sp, width):
    @pl.loop(0, RPT // ZR)
    def zcp(k):
        base = s * RPT + k * ZR
        for w in range(width // sw):
            pltpu.sync_copy(zbuf, acc_sp.at[pl.ds(base, ZR),
                                            pl.ds(w * sw, sw)])


def _splat(val):
    return jnp.full((16,), val, I32)


def _sc_layer1(src, dst, ree, pn, qn, v1):
    mesh = plsc.VectorSubcoreMesh(**_MESH)
    nbuf = [
        pltpu.VMEM((CH,), I32),
        pltpu.VMEM((CH,), I32),
        pltpu.VMEM((CH, 16), F32),
        pltpu.VMEM((CH, 16), F32),
        pltpu.VMEM((CH, 32), F32),
        pltpu.VMEM((CH, 128), F32),
    ]

    @functools.partial(
        pl.kernel,
        out_type=[jax.ShapeDtypeStruct((2, N, 128), F32),
                  jax.ShapeDtypeStruct((2, N, 48), F32)],
        mesh=mesh,
        compiler_params=pltpu.CompilerParams(needs_layout_passes=False,
                                             use_tc_tiling_on_sc=False),
        scratch_types=[
            pltpu.VMEM_SHARED((N, 128), F32),
            pltpu.VMEM_SHARED((N, 48), F32),
        ] + nbuf + nbuf + [
            pltpu.VMEM((CH, 48), F32),
            pltpu.VMEM((CH * 16,), F32),
            pltpu.VMEM((ZR, 64), F32),
            pltpu.VMEM((ZR, 16), F32),
            pltpu.SemaphoreType.DMA,
            pltpu.SemaphoreType.DMA,
            pltpu.SemaphoreType.DMA,
            pltpu.SemaphoreType.DMA,
        ],
    )
    def body(src_h, dst_h, ree_h, pn_h, qn_h, v1_h,
             m_out, rg_out,
             m_sp, rg_sp,
             srcA, dstA, gpA, gqA, reA, vrA,
             srcB, dstB, gpB, gqB, reB, vrB,
             rgrows, ef, zb64, zb16,
             slA, slB, sgA, sgB):
        c = lax.axis_index("c")
        s = lax.axis_index("s")
        wid = s * 2 + c
        _zero_fill(zb64, 64)
        _zero_fill(zb16, 16)
        _zero_acc(zb64, 64, s, m_sp, 128)
        _zero_acc(zb16, 16, s, rg_sp, 48)
        plsc.subcore_barrier()

        def lin_issue(i, bufs, sem):
            off = wid * EPW + jnp.minimum(i, NCHUNK - 1) * CH
            pltpu.async_copy(src_h.at[pl.ds(off, CH)], bufs[0], sem)
            pltpu.async_copy(dst_h.at[pl.ds(off, CH)], bufs[1], sem)
            pltpu.async_copy(ree_h.at[pl.ds(off, CH)], bufs[4], sem)

        def lin_wait(bufs, sem):
            pltpu.make_async_copy(src_h.at[pl.ds(0, CH)], bufs[0], sem).wait()
            pltpu.make_async_copy(dst_h.at[pl.ds(0, CH)], bufs[1], sem).wait()
            pltpu.make_async_copy(ree_h.at[pl.ds(0, CH)], bufs[4], sem).wait()

        def gath_issue(bufs, sem):
            pltpu.async_copy(pn_h.at[bufs[0]], bufs[2], sem)
            pltpu.async_copy(qn_h.at[bufs[1]], bufs[3], sem)
            pltpu.async_copy(v1_h.at[bufs[1]], bufs[5], sem)

        def gath_wait(bufs, sem):
            pltpu.make_async_copy(pn_h.at[bufs[0]], bufs[2], sem).wait()
            pltpu.make_async_copy(qn_h.at[bufs[1]], bufs[3], sem).wait()
            pltpu.make_async_copy(v1_h.at[bufs[1]], bufs[5], sem).wait()

        def compute(bufs):
            srcv, dstv, gpv, gqv, reev, vrowsv = bufs

            @pl.loop(0, CH, unroll=4)
            def ecomp(j):
                sv = gpv[j] + gqv[j] + reev[j, pl.ds(0, 16)]
                e = jnp.exp(-jnp.maximum(sv, 0.2 * sv))
                rgrows[j, pl.ds(0, 16)] = e
                ef[pl.ds(j * 16, 16)] = e

            @pl.loop(0, CH, unroll=4)
            def tcomp(cc):
                e0 = plsc.load_gather(ef, [_splat(cc * 16)])
                e1 = plsc.load_gather(ef, [_splat(cc * 16 + 1)])
                eerow = reev[cc, pl.ds(16, 16)]
                rgrows[cc, pl.ds(16, 16)] = e0 * eerow
                rgrows[cc, pl.ds(32, 16)] = e1 * eerow
                for jj in range(8):
                    eh = e0 if jj < 4 else e1
                    sl = pl.ds(jj * 16, 16)
                    vrowsv[cc, sl] = eh * vrowsv[cc, sl]

            pltpu.sync_copy(rgrows, rg_sp.at[srcv], add=True)
            pltpu.sync_copy(vrowsv, m_sp.at[srcv], add=True)

        A = (srcA, dstA, gpA, gqA, reA, vrA)
        B = (srcB, dstB, gpB, gqB, reB, vrB)

        lin_issue(0, A, slA)
        lin_wait(A, slA)
        gath_issue(A, sgA)
        lin_issue(1, B, slB)

        @pl.loop(0, NCHUNK // 2)
        def it(k):
            lin_wait(B, slB)
            gath_issue(B, sgB)
            gath_wait(A, sgA)
            compute(A)
            lin_issue(2 * k + 2, A, slA)
            lin_wait(A, slA)
            gath_issue(A, sgA)
            gath_wait(B, sgB)
            compute(B)
            lin_issue(2 * k + 3, B, slB)

        gath_wait(A, sgA)
        if NCHUNK % 2 == 1:
            compute(A)
        lin_wait(B, slB)

        plsc.subcore_barrier()

        @pl.when(s == 0)
        def _():
            pltpu.sync_copy(m_sp, m_out.at[c])
            pltpu.sync_copy(rg_sp, rg_out.at[c])

    return body(src, dst, ree, pn, qn, v1)


def _sc_layer2(src, dst, etyp, p2, q2, v2, relall):
    mesh = plsc.VectorSubcoreMesh(**_MESH)
    nbuf = [
        pltpu.VMEM((CH,), I32),
        pltpu.VMEM((CH,), I32),
        pltpu.VMEM((CH,), I32),
        pltpu.VMEM((CH, 16), F32),
        pltpu.VMEM((CH, 16), F32),
        pltpu.VMEM((CH, 128), F32),
    ]

    @functools.partial(
        pl.kernel,
        out_type=[jax.ShapeDtypeStruct((2, N, 128), F32),
                  jax.ShapeDtypeStruct((2, N, 16), F32)],
        mesh=mesh,
        compiler_params=pltpu.CompilerParams(needs_layout_passes=False,
                                             use_tc_tiling_on_sc=False),
        scratch_types=[
            pltpu.VMEM_SHARED((N, 128), F32),
            pltpu.VMEM_SHARED((N, 16), F32),
            pltpu.VMEM((200 * 129,), F32),
        ] + nbuf + nbuf + [
            pltpu.VMEM((CH, 16), F32),
            pltpu.VMEM((CH * 16,), F32),
            pltpu.VMEM((ZR, 16), F32),
            pltpu.SemaphoreType.DMA,
            pltpu.SemaphoreType.DMA,
            pltpu.SemaphoreType.DMA,
            pltpu.SemaphoreType.DMA,
        ],
    )
    def body(src_h, dst_h, typ_h, p2_h, q2_h, v2_h, rp_h,
             m_out, rs_out,
             m_sp, rs_sp, relv,
             srcA, dstA, typA, gpA, gqA, vrA,
             srcB, dstB, typB, gpB, gqB, vrB,
             ev, ef, zb16,
             slA, slB, sgA, sgB):
        c = lax.axis_index("c")
        s = lax.axis_index("s")
        wid = s * 2 + c
        _zero_fill(zb16, 16)

        @pl.loop(0, 25)
        def rcp(k):
            sl = pl.ds(k * 1032, 1032)
            pltpu.sync_copy(rp_h.at[sl], relv.at[sl])

        _zero_acc(zb16, 16, s, m_sp, 128)
        _zero_acc(zb16, 16, s, rs_sp, 16)
        plsc.subcore_barrier()

        def lin_issue(i, bufs, sem):
            off = wid * EPW + jnp.minimum(i, NCHUNK - 1) * CH
            pltpu.async_copy(src_h.at[pl.ds(off, CH)], bufs[0], sem)
            pltpu.async_copy(dst_h.at[pl.ds(off, CH)], bufs[1], sem)
            pltpu.async_copy(typ_h.at[pl.ds(off, CH)], bufs[2], sem)

        def lin_wait(bufs, sem):
            pltpu.make_async_copy(src_h.at[pl.ds(0, CH)], bufs[0], sem).wait()
            pltpu.make_async_copy(dst_h.at[pl.ds(0, CH)], bufs[1], sem).wait()
            pltpu.make_async_copy(typ_h.at[pl.ds(0, CH)], bufs[2], sem).wait()

        def gath_issue(bufs, sem):
            pltpu.async_copy(p2_h.at[bufs[0]], bufs[3], sem)
            pltpu.async_copy(q2_h.at[bufs[1]], bufs[4], sem)
            pltpu.async_copy(v2_h.at[bufs[1]], bufs[5], sem)

        def gath_wait(bufs, sem):
            pltpu.make_async_copy(p2_h.at[bufs[0]], bufs[3], sem).wait()
            pltpu.make_async_copy(q2_h.at[bufs[1]], bufs[4], sem).wait()
            pltpu.make_async_copy(v2_h.at[bufs[1]], bufs[5], sem).wait()

        lanes = lax.iota(I32, 16)

        def compute(bufs):
            srcv, dstv, typv, gpv, gqv, vrowsv = bufs

            @pl.loop(0, CH, unroll=8)
            def ecomp(j):
                tj = plsc.load_gather(typv, [_splat(j)])
                rr = plsc.load_gather(relv, [tj + 200 * 128])
                sv = gpv[j] + gqv[j] + rr
                e = jnp.exp(-jnp.maximum(sv, 0.2 * sv))
                ev[j] = e
                ef[pl.ds(j * 16, 16)] = e

            pltpu.sync_copy(ev, rs_sp.at[srcv], add=True)

            @pl.loop(0, CH, unroll=8)
            def tcomp(cc):
                e0 = plsc.load_gather(ef, [_splat(cc * 16)])
                tvec = plsc.load_gather(typv, [_splat(cc)])
                tbase = tvec * 128
                for jj in range(8):
                    sl = pl.ds(jj * 16, 16)
                    w2j = plsc.load_gather(relv, [tbase + lanes + jj * 16])
                    vrowsv[cc, sl] = e0 * (vrowsv[cc, sl] + w2j)

            pltpu.sync_copy(vrowsv, m_sp.at[srcv], add=True)

        A = (srcA, dstA, typA, gpA, gqA, vrA)
        B = (srcB, dstB, typB, gpB, gqB, vrB)

        lin_issue(0, A, slA)
        lin_wait(A, slA)
        gath_issue(A, sgA)
        lin_issue(1, B, slB)

        @pl.loop(0, NCHUNK // 2)
        def it(k):
            lin_wait(B, slB)
            gath_issue(B, sgB)
            gath_wait(A, sgA)
            compute(A)
            lin_issue(2 * k + 2, A, slA)
            lin_wait(A, slA)
            gath_issue(A, sgA)
            gath_wait(B, sgB)
            compute(B)
            lin_issue(2 * k + 3, B, slB)

        gath_wait(A, sgA)
        if NCHUNK % 2 == 1:
            compute(A)
        lin_wait(B, slB)

        plsc.subcore_barrier()

        @pl.when(s == 0)
        def _():
            pltpu.sync_copy(m_sp, m_out.at[c])
            pltpu.sync_copy(rs_sp, rs_out.at[c])

    return body(src, dst, etyp, p2, q2, v2, relall)


def kernel(Corpus_, batch_inputs, unique_entity_embed, unique_relation_embed,
           edge_list, edge_type, edge_embed, a0, a2_0, a1, a2_1, W_1, a_out,
           a2_out):
    x = unique_entity_embed.astype(F32)
    ee = edge_embed.astype(F32)
    src = edge_list[0].astype(I32)
    dst = edge_list[1].astype(I32)
    etyp = edge_type.astype(I32)

    ast = jnp.concatenate([a0[:, :F], a1[:, :F]], axis=0).T
    adt = jnp.concatenate([a0[:, F:2 * F], a1[:, F:2 * F]], axis=0).T
    a2p = jnp.zeros((128, 16), F32)
    a2p = a2p.at[0:64, 0].set(a2_0[0]).at[64:128, 1].set(a2_1[0])
    ar0 = (a2_0 @ a0[:, 2 * F:2 * F + 16])[0]
    ar1 = (a2_1 @ a1[:, 2 * F:2 * F + 16])[0]
    ar16 = jnp.zeros((16, 16), F32).at[:, 0].set(ar0).at[:, 1].set(ar1)
    kmat = jnp.zeros((32, 128), F32)
    kmat = kmat.at[0:16, 0:64].set(a0[:, 2 * F:2 * F + 16].T)
    kmat = kmat.at[16:32, 64:128].set(a1[:, 2 * F:2 * F + 16].T)
    a2st = a_out[:, :128].T
    a2dt = a_out[:, 128:256].T
    a2rt = a_out[:, 256:384].T
    a2p2 = jnp.zeros((128, 16), F32).at[:, 0].set(a2_out[0])

    u, v1, pn, qn = _proj_nodes(x, ast, adt, a2p)
    ree = _ree(ee, ar16)

    m1, rg1 = _sc_layer1(src, dst, ree, pn, qn, v1)

    relproj, relr = _relproj(unique_relation_embed.astype(F32), W_1, a2rt,
                             a2p2)
    relall = jnp.concatenate([relproj.reshape(-1), relr[:, 0]])

    u2, v2, p2, q2 = _combine1(u, m1, rg1, kmat, a2st, a2dt, a2p2)

    m2, rs2 = _sc_layer2(src, dst, etyp, p2, q2, v2, relall)

    return _final(u2, m2, rs2)
